# baseline (speedup 1.0000x reference)
import functools

import numpy as np

import jax
import jax.numpy as jnp
from jax import lax
from jax.experimental import pallas as pl
from jax.experimental.pallas import tpu as pltpu
from jax.experimental.pallas import tpu_sc as plsc

B = 8
N = 20000
G = 100
NREAL = N + G
NP = 20480
NSUB = NP // 128
NUM_FG = 128
NUM_BG = 384
NSEL = NUM_FG + NUM_BG
FG_THRESH = 0.5

_KEY_BIAS = np.int32(0x3F810000)
_INVALID24 = np.int32(0x00FFFFFF)


def _stage_a_body(gt_ref, x1_ref, y1_ref, x2_ref, y2_ref, miou_ref, amax_ref):
    x1 = x1_ref[0]
    y1 = y1_ref[0]
    x2 = x2_ref[0]
    y2 = y2_ref[0]
    area = (x2 - x1) * (y2 - y1)

    def body(g, carry):
        best, bidx = carry
        gx1 = gt_ref[0, g, 0]
        gy1 = gt_ref[0, g, 1]
        gx2 = gt_ref[0, g, 2]
        gy2 = gt_ref[0, g, 3]
        garea = (gx2 - gx1) * (gy2 - gy1)
        iw = jnp.maximum(jnp.minimum(x2, gx2) - jnp.maximum(x1, gx1), 0.0)
        ih = jnp.maximum(jnp.minimum(y2, gy2) - jnp.maximum(y1, gy1), 0.0)
        inter = iw * ih
        union = jnp.maximum(area + garea - inter, 1e-8)
        q = inter / union
        upd = q > best
        best = jnp.where(upd, q, best)
        bidx = jnp.where(upd, g, bidx)
        return best, bidx

    best0 = jnp.zeros_like(x1)
    bidx0 = jnp.zeros(x1.shape, jnp.int32)
    best, bidx = lax.fori_loop(0, G, body, (best0, bidx0))
    miou_ref[0] = best
    amax_ref[0] = bidx


def _stage_a(gt_boxes, x1, y1, x2, y2):
    chunk = 40
    grid = (B, NSUB // chunk)
    plane_spec = pl.BlockSpec((1, chunk, 128), lambda b, c: (b, c, 0))
    return pl.pallas_call(
        _stage_a_body,
        grid=grid,
        in_specs=[
            pl.BlockSpec((1, G, 4), lambda b, c: (b, 0, 0),
                         memory_space=pltpu.SMEM),
            plane_spec, plane_spec, plane_spec, plane_spec,
        ],
        out_specs=[plane_spec, plane_spec],
        out_shape=[
            jax.ShapeDtypeStruct((B, NSUB, 128), jnp.float32),
            jax.ShapeDtypeStruct((B, NSUB, 128), jnp.int32),
        ],
    )(gt_boxes, x1, y1, x2, y2)


def _stage_b_body(miou_hbm, amax_hbm,
                  bx1_hbm, by1_hbm, bx2_hbm, by2_hbm,
                  gx1_hbm, gy1_hbm, gx2_hbm, gy2_hbm, labels_hbm,
                  rx1_hbm, ry1_hbm, rx2_hbm, ry2_hbm,
                  tx1_hbm, ty1_hbm, tx2_hbm, ty2_hbm,
                  lblout_hbm, isfg_hbm,
                  kf, ki, idx_a, idx_b, hist, offs, carry16,
                  idx_sel, am_v, gti_v, val_v, lbl_v, isfg_v, sem):
    c = lax.axis_index("c")
    s = lax.axis_index("s")
    wid = s * 2 + c
    active = wid < 16
    img = wid // 2
    cl = wid % 2

    @pl.when(active)
    def _():
        iota16 = lax.iota(jnp.int32, 16)
        lane15 = jnp.full((16,), 15, jnp.int32)

        pltpu.sync_copy(miou_hbm.at[pl.ds(img * NP, NP)], kf)

        def key_body(i, _):
            m = kf[pl.ds(i * 16, 16)]
            valid = (iota16 + i * 16) < NREAL
            fg = jnp.where(valid & (m >= FG_THRESH), m, -1.0)
            bg = jnp.where(valid & (m < FG_THRESH), 1.0 - m, -1.0)
            score = jnp.where(cl == 0, fg, bg)
            sb = lax.bitcast_convert_type(score, jnp.int32)
            inv = jnp.where(sb >= 0, _KEY_BIAS - sb, _INVALID24)
            kf[pl.ds(i * 16, 16)] = lax.bitcast_convert_type(inv, jnp.float32)
            return 0

        lax.fori_loop(0, NP // 16, key_body, 0)

        def radix_pass(shift, src_key, src_f32, dst_key, dst_f32,
                       src_idx, dst_idx, first):
            def load_key(i):
                k = src_key[pl.ds(i * 16, 16)]
                return lax.bitcast_convert_type(k, jnp.int32) if src_f32 else k

            zero16 = jnp.zeros((16,), jnp.int32)
            for j in range(16):
                hist[pl.ds(j * 16, 16)] = zero16

            shift_v = jnp.full((16,), shift, jnp.int32)

            def h_body(i, _):
                kb = load_key(i)
                d = lax.shift_right_logical(kb, shift_v) & jnp.int32(255)
                cnt, last = plsc.scan_count(d)
                plsc.addupdate_scatter(hist, [d], cnt, mask=last)
                return 0

            lax.fori_loop(0, NP // 16, h_body, 0)

            def p_body(j, carry):
                v = hist[pl.ds(j * 16, 16)]
                csum = plsc.cumsum(v)
                offs[pl.ds(j * 16, 16)] = csum - v + carry
                carry16[...] = csum
                tot = plsc.load_gather(carry16, [lane15])
                return carry + tot

            lax.fori_loop(0, 16, p_body, jnp.zeros((16,), jnp.int32))

            def s_body(i, _):
                kb = load_key(i)
                d = lax.shift_right_logical(kb, shift_v) & jnp.int32(255)
                if first:
                    iv = iota16 + i * 16
                else:
                    iv = src_idx[pl.ds(i * 16, 16)]
                base = plsc.load_gather(offs, [d])
                cnt, last = plsc.scan_count(d)
                pos = base + cnt - 1
                kout = lax.bitcast_convert_type(kb, jnp.float32) if dst_f32 else kb
                plsc.store_scatter(dst_key, [pos], kout)
                plsc.store_scatter(dst_idx, [pos], iv)
                plsc.addupdate_scatter(offs, [d], cnt, mask=last)
                return 0

            lax.fori_loop(0, NP // 16, s_body, 0)

        radix_pass(0, kf, True, ki, False, idx_a, idx_b, True)
        radix_pass(8, ki, False, kf, True, idx_b, idx_a, False)
        radix_pass(16, kf, True, ki, False, idx_a, idx_b, False)

        def emit(k_count, slot0, is_fg_unit):
            nv = k_count // 16
            nch = k_count // 128

            def sel_body(i, _):
                iv = idx_b[pl.ds(i * 16, 16)]
                row = i >> 3
                col = (i & 7) * 16
                idx_sel[row, pl.ds(col, 16)] = iv + img * NP
                if is_fg_unit:
                    inv = ki[pl.ds(i * 16, 16)]
                    fgm = inv < _INVALID24
                    isfg_v[pl.ds(i * 16, 16)] = jnp.where(fgm, 1, 0)
                else:
                    isfg_v[pl.ds(i * 16, 16)] = jnp.zeros((16,), jnp.int32)
                return 0

            lax.fori_loop(0, nv, sel_body, 0)

            out0 = img * NSEL + slot0

            for j in range(nch):
                pltpu.async_copy(amax_hbm.at[idx_sel.at[j]],
                                 am_v.at[j], sem).wait()

            def gti_body(i, _):
                row = i >> 3
                col = (i & 7) * 16
                am = am_v[row, pl.ds(col, 16)]
                gti_v[row, pl.ds(col, 16)] = am + img * G
                return 0

            lax.fori_loop(0, nv, gti_body, 0)

            for src, dst in ((bx1_hbm, rx1_hbm), (by1_hbm, ry1_hbm),
                             (bx2_hbm, rx2_hbm), (by2_hbm, ry2_hbm)):
                for j in range(nch):
                    pltpu.async_copy(src.at[idx_sel.at[j]],
                                     val_v.at[pl.ds(j * 128, 128)], sem).wait()
                pltpu.sync_copy(val_v.at[pl.ds(0, k_count)],
                                dst.at[pl.ds(out0, k_count)])
            for src, dst in ((gx1_hbm, tx1_hbm), (gy1_hbm, ty1_hbm),
                             (gx2_hbm, tx2_hbm), (gy2_hbm, ty2_hbm)):
                for j in range(nch):
                    pltpu.async_copy(src.at[gti_v.at[j]],
                                     val_v.at[pl.ds(j * 128, 128)], sem).wait()
                pltpu.sync_copy(val_v.at[pl.ds(0, k_count)],
                                dst.at[pl.ds(out0, k_count)])
            for j in range(nch):
                pltpu.async_copy(labels_hbm.at[gti_v.at[j]],
                                 lbl_v.at[pl.ds(j * 128, 128)], sem).wait()

            pltpu.sync_copy(lbl_v.at[pl.ds(0, k_count)],
                            lblout_hbm.at[pl.ds(out0, k_count)])
            pltpu.sync_copy(isfg_v.at[pl.ds(0, k_count)],
                            isfg_hbm.at[pl.ds(out0, k_count)])

        @pl.when(cl == 0)
        def _():
            emit(NUM_FG, 0, True)

        @pl.when(cl == 1)
        def _():
            emit(NUM_BG, NUM_FG, False)


def _stage_b(miou_flat, amax_flat, box_planes, gt_planes, labels_flat):
    mesh = plsc.VectorSubcoreMesh(core_axis_name="c", subcore_axis_name="s")
    fvec = jax.ShapeDtypeStruct((B * NSEL,), jnp.float32)
    ivec = jax.ShapeDtypeStruct((B * NSEL,), jnp.int32)
    f = pl.kernel(
        _stage_b_body,
        out_type=[fvec, fvec, fvec, fvec,
                  fvec, fvec, fvec, fvec,
                  ivec, ivec],
        mesh=mesh,
        compiler_params=pltpu.CompilerParams(needs_layout_passes=False),
        scratch_types=[
            pltpu.VMEM((NP,), jnp.float32),
            pltpu.VMEM((NP,), jnp.int32),
            pltpu.VMEM((NP,), jnp.int32),
            pltpu.VMEM((NP,), jnp.int32),
            pltpu.VMEM((256,), jnp.int32),
            pltpu.VMEM((256,), jnp.int32),
            pltpu.VMEM((16,), jnp.int32),
            pltpu.VMEM((3, 128), jnp.int32),
            pltpu.VMEM((3, 128), jnp.int32),
            pltpu.VMEM((3, 128), jnp.int32),
            pltpu.VMEM((384,), jnp.float32),
            pltpu.VMEM((384,), jnp.int32),
            pltpu.VMEM((384,), jnp.int32),
            pltpu.SemaphoreType.DMA,
        ],
    )
    return f(miou_flat, amax_flat, *box_planes, *gt_planes, labels_flat)


def _stage_c_body(roist_ref, gtbt_ref, lbl_ref, isfg_ref, btt_ref, cls_ref):
    rx1 = roist_ref[0]
    ry1 = roist_ref[1]
    rx2 = roist_ref[2]
    ry2 = roist_ref[3]
    gx1 = gtbt_ref[0]
    gy1 = gtbt_ref[1]
    gx2 = gtbt_ref[2]
    gy2 = gtbt_ref[3]
    pw = jnp.maximum(rx2 - rx1, 1e-8)
    ph = jnp.maximum(ry2 - ry1, 1e-8)
    px = rx1 + 0.5 * pw
    py = ry1 + 0.5 * ph
    gw = jnp.maximum(gx2 - gx1, 1e-8)
    gh = jnp.maximum(gy2 - gy1, 1e-8)
    gx = gx1 + 0.5 * gw
    gy = gy1 + 0.5 * gh
    fg = isfg_ref[...] != 0
    zero = jnp.zeros_like(pw)
    btt_ref[0] = jnp.where(fg, (gx - px) / pw, zero)
    btt_ref[1] = jnp.where(fg, (gy - py) / ph, zero)
    btt_ref[2] = jnp.where(fg, jnp.log(gw / pw), zero)
    btt_ref[3] = jnp.where(fg, jnp.log(gh / ph), zero)
    cls_ref[...] = jnp.where(fg, lbl_ref[...], 0)


def _stage_c(rois_t, gtb_t, lbl, isfg):
    return pl.pallas_call(
        _stage_c_body,
        out_shape=[
            jax.ShapeDtypeStruct((4, 32, 128), jnp.float32),
            jax.ShapeDtypeStruct((32, 128), jnp.int32),
        ],
    )(rois_t, gtb_t, lbl, isfg)


@jax.jit
def kernel(boxes, gt_boxes, gt_labels):
    boxes = lax.stop_gradient(boxes)
    all_boxes = jnp.concatenate([boxes, gt_boxes], axis=1)
    ab_pad = jnp.pad(all_boxes, ((0, 0), (0, NP - NREAL), (0, 0)))
    planes = ab_pad.transpose(0, 2, 1).reshape(B, 4, NSUB, 128)
    x1 = planes[:, 0]
    y1 = planes[:, 1]
    x2 = planes[:, 2]
    y2 = planes[:, 3]

    miou, amax = _stage_a(gt_boxes, x1, y1, x2, y2)

    box_planes = [planes[:, c].reshape(B * NP) for c in range(4)]
    gt_planes = [gt_boxes[:, :, c].reshape(B * G) for c in range(4)]
    outs = _stage_b(
        miou.reshape(B * NP),
        amax.reshape(B * NP),
        box_planes,
        gt_planes,
        gt_labels.reshape(B * G),
    )
    rois_p = outs[0:4]
    gtb_p = outs[4:8]
    lbl_f, isfg_f = outs[8], outs[9]

    rois_t = jnp.stack(rois_p).reshape(4, 32, 128)
    gtb_t = jnp.stack(gtb_p).reshape(4, 32, 128)
    btt, cls = _stage_c(rois_t, gtb_t,
                        lbl_f.reshape(32, 128), isfg_f.reshape(32, 128))

    rois = jnp.stack(rois_p, axis=-1).reshape(B, NSEL, 4)
    box_targets = btt.reshape(4, B * NSEL).T.reshape(B, NSEL, 4)
    class_targets = cls.reshape(B, NSEL)
    return rois, box_targets, class_targets


# speedup vs baseline: 1.1529x; 1.1529x over previous
import functools

import numpy as np

import jax
import jax.numpy as jnp
from jax import lax
from jax.experimental import pallas as pl
from jax.experimental.pallas import tpu as pltpu
from jax.experimental.pallas import tpu_sc as plsc

B = 8
N = 20000
G = 100
NREAL = N + G
NP = 20480
NSUB = NP // 128
NUM_FG = 128
NUM_BG = 384
NSEL = NUM_FG + NUM_BG
FG_THRESH = 0.5
NBKT = 4096

_KEY_BIAS = np.int32(0x3F810000)
_INVALID24 = np.int32(0x00FFFFFF)


def _stage_a_body(gt_ref, x1_ref, y1_ref, x2_ref, y2_ref, miou_ref, amax_ref):
    x1 = x1_ref[0]
    y1 = y1_ref[0]
    x2 = x2_ref[0]
    y2 = y2_ref[0]
    area = (x2 - x1) * (y2 - y1)

    def body(g, carry):
        best, bidx = carry
        gx1 = gt_ref[0, g, 0]
        gy1 = gt_ref[0, g, 1]
        gx2 = gt_ref[0, g, 2]
        gy2 = gt_ref[0, g, 3]
        garea = (gx2 - gx1) * (gy2 - gy1)
        iw = jnp.maximum(jnp.minimum(x2, gx2) - jnp.maximum(x1, gx1), 0.0)
        ih = jnp.maximum(jnp.minimum(y2, gy2) - jnp.maximum(y1, gy1), 0.0)
        inter = iw * ih
        union = jnp.maximum(area + garea - inter, 1e-8)
        q = inter / union
        upd = q > best
        best = jnp.where(upd, q, best)
        bidx = jnp.where(upd, g, bidx)
        return best, bidx

    best0 = jnp.zeros_like(x1)
    bidx0 = jnp.zeros(x1.shape, jnp.int32)
    best, bidx = lax.fori_loop(0, G, body, (best0, bidx0))
    miou_ref[0] = best
    amax_ref[0] = bidx


def _stage_a(gt_boxes, x1, y1, x2, y2):
    chunk = 40
    grid = (B, NSUB // chunk)
    plane_spec = pl.BlockSpec((1, chunk, 128), lambda b, c: (b, c, 0))
    return pl.pallas_call(
        _stage_a_body,
        grid=grid,
        in_specs=[
            pl.BlockSpec((1, G, 4), lambda b, c: (b, 0, 0),
                         memory_space=pltpu.SMEM),
            plane_spec, plane_spec, plane_spec, plane_spec,
        ],
        out_specs=[plane_spec, plane_spec],
        out_shape=[
            jax.ShapeDtypeStruct((B, NSUB, 128), jnp.float32),
            jax.ShapeDtypeStruct((B, NSUB, 128), jnp.int32),
        ],
    )(gt_boxes, x1, y1, x2, y2)


def _stage_b_body(miou_hbm, amax_hbm,
                  bx1_hbm, by1_hbm, bx2_hbm, by2_hbm,
                  gx1_hbm, gy1_hbm, gx2_hbm, gy2_hbm, labels_hbm,
                  rx1_hbm, ry1_hbm, rx2_hbm, ry2_hbm,
                  tx1_hbm, ty1_hbm, tx2_hbm, ty2_hbm,
                  lblout_hbm, isfg_hbm,
                  kf, ki, idx_a, idx_b, hist, offs, carry16,
                  idx_sel, am_v, gti_v, val_v, lbl_v, isfg_v, sem):
    c = lax.axis_index("c")
    s = lax.axis_index("s")
    wid = s * 2 + c
    active = wid < 16
    img = wid // 2
    cl = wid % 2

    @pl.when(active)
    def _():
        iota16 = lax.iota(jnp.int32, 16)
        lane15 = jnp.full((16,), 15, jnp.int32)

        pltpu.sync_copy(miou_hbm.at[pl.ds(img * NP, NP)], kf)

        def key_body(i, _):
            m = kf[pl.ds(i * 16, 16)]
            valid = (iota16 + i * 16) < NREAL
            fg = jnp.where(valid & (m >= FG_THRESH), m, -1.0)
            bg = jnp.where(valid & (m < FG_THRESH), 1.0 - m, -1.0)
            score = jnp.where(cl == 0, fg, bg)
            sb = lax.bitcast_convert_type(score, jnp.int32)
            inv = jnp.where(sb >= 0, _KEY_BIAS - sb, _INVALID24)
            kf[pl.ds(i * 16, 16)] = lax.bitcast_convert_type(inv, jnp.float32)
            return 0

        lax.fori_loop(0, NP // 16, key_body, 0)

        def radix_pass(shift, src_key, src_f32, dst_key, dst_f32,
                       src_idx, dst_idx, first):
            def load_key(i):
                k = src_key[pl.ds(i * 16, 16)]
                return lax.bitcast_convert_type(k, jnp.int32) if src_f32 else k

            zero16 = jnp.zeros((16,), jnp.int32)

            def z_body(j, _):
                hist[pl.ds(j * 16, 16)] = zero16
                return 0

            lax.fori_loop(0, NBKT // 16, z_body, 0)

            shift_v = jnp.full((16,), shift, jnp.int32)

            def h_body(i, _):
                kb = load_key(i)
                d = lax.shift_right_logical(kb, shift_v) & jnp.int32(NBKT - 1)
                cnt, last = plsc.scan_count(d)
                plsc.addupdate_scatter(hist, [d], cnt, mask=last)
                return 0

            lax.fori_loop(0, NP // 16, h_body, 0)

            def p_body(j, carry):
                v = hist[pl.ds(j * 16, 16)]
                csum = plsc.cumsum(v)
                offs[pl.ds(j * 16, 16)] = csum - v + carry
                carry16[...] = csum
                tot = plsc.load_gather(carry16, [lane15])
                return carry + tot

            lax.fori_loop(0, NBKT // 16, p_body, jnp.zeros((16,), jnp.int32))

            def s_body(i, _):
                kb = load_key(i)
                d = lax.shift_right_logical(kb, shift_v) & jnp.int32(NBKT - 1)
                if first:
                    iv = iota16 + i * 16
                else:
                    iv = src_idx[pl.ds(i * 16, 16)]
                base = plsc.load_gather(offs, [d])
                cnt, last = plsc.scan_count(d)
                pos = base + cnt - 1
                kout = lax.bitcast_convert_type(kb, jnp.float32) if dst_f32 else kb
                plsc.store_scatter(dst_key, [pos], kout)
                plsc.store_scatter(dst_idx, [pos], iv)
                plsc.addupdate_scatter(offs, [d], cnt, mask=last)
                return 0

            lax.fori_loop(0, NP // 16, s_body, 0)

        radix_pass(0, kf, True, ki, False, idx_a, idx_b, True)
        radix_pass(12, ki, False, kf, True, idx_b, idx_a, False)

        def emit(k_count, slot0, is_fg_unit):
            nv = k_count // 16
            nch = k_count // 128

            def sel_body(i, _):
                iv = idx_a[pl.ds(i * 16, 16)]
                row = i >> 3
                col = (i & 7) * 16
                idx_sel[row, pl.ds(col, 16)] = iv + img * NP
                if is_fg_unit:
                    inv = lax.bitcast_convert_type(kf[pl.ds(i * 16, 16)],
                                                   jnp.int32)
                    fgm = inv < _INVALID24
                    isfg_v[pl.ds(i * 16, 16)] = jnp.where(fgm, 1, 0)
                else:
                    isfg_v[pl.ds(i * 16, 16)] = jnp.zeros((16,), jnp.int32)
                return 0

            lax.fori_loop(0, nv, sel_body, 0)

            out0 = img * NSEL + slot0

            for j in range(nch):
                pltpu.async_copy(amax_hbm.at[idx_sel.at[j]],
                                 am_v.at[j], sem).wait()

            def gti_body(i, _):
                row = i >> 3
                col = (i & 7) * 16
                am = am_v[row, pl.ds(col, 16)]
                gti_v[row, pl.ds(col, 16)] = am + img * G
                return 0

            lax.fori_loop(0, nv, gti_body, 0)

            for src, dst in ((bx1_hbm, rx1_hbm), (by1_hbm, ry1_hbm),
                             (bx2_hbm, rx2_hbm), (by2_hbm, ry2_hbm)):
                for j in range(nch):
                    pltpu.async_copy(src.at[idx_sel.at[j]],
                                     val_v.at[pl.ds(j * 128, 128)], sem).wait()
                pltpu.sync_copy(val_v.at[pl.ds(0, k_count)],
                                dst.at[pl.ds(out0, k_count)])
            for src, dst in ((gx1_hbm, tx1_hbm), (gy1_hbm, ty1_hbm),
                             (gx2_hbm, tx2_hbm), (gy2_hbm, ty2_hbm)):
                for j in range(nch):
                    pltpu.async_copy(src.at[gti_v.at[j]],
                                     val_v.at[pl.ds(j * 128, 128)], sem).wait()
                pltpu.sync_copy(val_v.at[pl.ds(0, k_count)],
                                dst.at[pl.ds(out0, k_count)])
            for j in range(nch):
                pltpu.async_copy(labels_hbm.at[gti_v.at[j]],
                                 lbl_v.at[pl.ds(j * 128, 128)], sem).wait()

            pltpu.sync_copy(lbl_v.at[pl.ds(0, k_count)],
                            lblout_hbm.at[pl.ds(out0, k_count)])
            pltpu.sync_copy(isfg_v.at[pl.ds(0, k_count)],
                            isfg_hbm.at[pl.ds(out0, k_count)])

        @pl.when(cl == 0)
        def _():
            emit(NUM_FG, 0, True)

        @pl.when(cl == 1)
        def _():
            emit(NUM_BG, NUM_FG, False)


def _stage_b(miou_flat, amax_flat, box_planes, gt_planes, labels_flat):
    mesh = plsc.VectorSubcoreMesh(core_axis_name="c", subcore_axis_name="s")
    fvec = jax.ShapeDtypeStruct((B * NSEL,), jnp.float32)
    ivec = jax.ShapeDtypeStruct((B * NSEL,), jnp.int32)
    f = pl.kernel(
        _stage_b_body,
        out_type=[fvec, fvec, fvec, fvec,
                  fvec, fvec, fvec, fvec,
                  ivec, ivec],
        mesh=mesh,
        compiler_params=pltpu.CompilerParams(needs_layout_passes=False),
        scratch_types=[
            pltpu.VMEM((NP,), jnp.float32),
            pltpu.VMEM((NP,), jnp.int32),
            pltpu.VMEM((NP,), jnp.int32),
            pltpu.VMEM((NP,), jnp.int32),
            pltpu.VMEM((NBKT,), jnp.int32),
            pltpu.VMEM((NBKT,), jnp.int32),
            pltpu.VMEM((16,), jnp.int32),
            pltpu.VMEM((3, 128), jnp.int32),
            pltpu.VMEM((3, 128), jnp.int32),
            pltpu.VMEM((3, 128), jnp.int32),
            pltpu.VMEM((384,), jnp.float32),
            pltpu.VMEM((384,), jnp.int32),
            pltpu.VMEM((384,), jnp.int32),
            pltpu.SemaphoreType.DMA,
        ],
    )
    return f(miou_flat, amax_flat, *box_planes, *gt_planes, labels_flat)


def _stage_c_body(roist_ref, gtbt_ref, lbl_ref, isfg_ref, btt_ref, cls_ref):
    rx1 = roist_ref[0]
    ry1 = roist_ref[1]
    rx2 = roist_ref[2]
    ry2 = roist_ref[3]
    gx1 = gtbt_ref[0]
    gy1 = gtbt_ref[1]
    gx2 = gtbt_ref[2]
    gy2 = gtbt_ref[3]
    pw = jnp.maximum(rx2 - rx1, 1e-8)
    ph = jnp.maximum(ry2 - ry1, 1e-8)
    px = rx1 + 0.5 * pw
    py = ry1 + 0.5 * ph
    gw = jnp.maximum(gx2 - gx1, 1e-8)
    gh = jnp.maximum(gy2 - gy1, 1e-8)
    gx = gx1 + 0.5 * gw
    gy = gy1 + 0.5 * gh
    fg = isfg_ref[...] != 0
    zero = jnp.zeros_like(pw)
    btt_ref[0] = jnp.where(fg, (gx - px) / pw, zero)
    btt_ref[1] = jnp.where(fg, (gy - py) / ph, zero)
    btt_ref[2] = jnp.where(fg, jnp.log(gw / pw), zero)
    btt_ref[3] = jnp.where(fg, jnp.log(gh / ph), zero)
    cls_ref[...] = jnp.where(fg, lbl_ref[...], 0)


def _stage_c(rois_t, gtb_t, lbl, isfg):
    return pl.pallas_call(
        _stage_c_body,
        out_shape=[
            jax.ShapeDtypeStruct((4, 32, 128), jnp.float32),
            jax.ShapeDtypeStruct((32, 128), jnp.int32),
        ],
    )(rois_t, gtb_t, lbl, isfg)


@jax.jit
def kernel(boxes, gt_boxes, gt_labels):
    boxes = lax.stop_gradient(boxes)
    all_boxes = jnp.concatenate([boxes, gt_boxes], axis=1)
    ab_pad = jnp.pad(all_boxes, ((0, 0), (0, NP - NREAL), (0, 0)))
    planes = ab_pad.transpose(0, 2, 1).reshape(B, 4, NSUB, 128)
    x1 = planes[:, 0]
    y1 = planes[:, 1]
    x2 = planes[:, 2]
    y2 = planes[:, 3]

    miou, amax = _stage_a(gt_boxes, x1, y1, x2, y2)

    box_planes = [planes[:, c].reshape(B * NP) for c in range(4)]
    gt_planes = [gt_boxes[:, :, c].reshape(B * G) for c in range(4)]
    outs = _stage_b(
        miou.reshape(B * NP),
        amax.reshape(B * NP),
        box_planes,
        gt_planes,
        gt_labels.reshape(B * G),
    )
    rois_p = outs[0:4]
    gtb_p = outs[4:8]
    lbl_f, isfg_f = outs[8], outs[9]

    rois_t = jnp.stack(rois_p).reshape(4, 32, 128)
    gtb_t = jnp.stack(gtb_p).reshape(4, 32, 128)
    btt, cls = _stage_c(rois_t, gtb_t,
                        lbl_f.reshape(32, 128), isfg_f.reshape(32, 128))

    rois = jnp.stack(rois_p, axis=-1).reshape(B, NSEL, 4)
    box_targets = btt.reshape(4, B * NSEL).T.reshape(B, NSEL, 4)
    class_targets = cls.reshape(B, NSEL)
    return rois, box_targets, class_targets


# speedup vs baseline: 1.2829x; 1.1128x over previous
import functools

import numpy as np

import jax
import jax.numpy as jnp
from jax import lax
from jax.experimental import pallas as pl
from jax.experimental.pallas import tpu as pltpu
from jax.experimental.pallas import tpu_sc as plsc

B = 8
N = 20000
G = 100
NREAL = N + G
NP = 20480
NSUB = NP // 128
NUM_FG = 128
NUM_BG = 384
NSEL = NUM_FG + NUM_BG
FG_THRESH = 0.5
NBKT = 4096

_KEY_BIAS = np.int32(0x3F810000)
_INVALID24 = np.int32(0x00FFFFFF)


def _stage_a_body(gt_ref, x1_ref, y1_ref, x2_ref, y2_ref, miou_ref, amax_ref):
    x1 = x1_ref[0]
    y1 = y1_ref[0]
    x2 = x2_ref[0]
    y2 = y2_ref[0]
    area = (x2 - x1) * (y2 - y1)

    def body(g, carry):
        best, bidx = carry
        gx1 = gt_ref[0, g, 0]
        gy1 = gt_ref[0, g, 1]
        gx2 = gt_ref[0, g, 2]
        gy2 = gt_ref[0, g, 3]
        garea = (gx2 - gx1) * (gy2 - gy1)
        iw = jnp.maximum(jnp.minimum(x2, gx2) - jnp.maximum(x1, gx1), 0.0)
        ih = jnp.maximum(jnp.minimum(y2, gy2) - jnp.maximum(y1, gy1), 0.0)
        inter = iw * ih
        union = jnp.maximum(area + garea - inter, 1e-8)
        q = inter / union
        upd = q > best
        best = jnp.where(upd, q, best)
        bidx = jnp.where(upd, g, bidx)
        return best, bidx

    best0 = jnp.zeros_like(x1)
    bidx0 = jnp.zeros(x1.shape, jnp.int32)
    best, bidx = lax.fori_loop(0, G, body, (best0, bidx0))
    miou_ref[0] = best
    amax_ref[0] = bidx


def _stage_a(gt_boxes, x1, y1, x2, y2):
    chunk = 160
    grid = (B, NSUB // chunk)
    plane_spec = pl.BlockSpec((1, chunk, 128), lambda b, c: (b, c, 0))
    return pl.pallas_call(
        _stage_a_body,
        grid=grid,
        in_specs=[
            pl.BlockSpec((1, G, 4), lambda b, c: (b, 0, 0),
                         memory_space=pltpu.SMEM),
            plane_spec, plane_spec, plane_spec, plane_spec,
        ],
        out_specs=[plane_spec, plane_spec],
        out_shape=[
            jax.ShapeDtypeStruct((B, NSUB, 128), jnp.float32),
            jax.ShapeDtypeStruct((B, NSUB, 128), jnp.int32),
        ],
    )(gt_boxes, x1, y1, x2, y2)


def _stage_b_body(miou_hbm, amax_hbm,
                  bx1_hbm, by1_hbm, bx2_hbm, by2_hbm,
                  gx1_hbm, gy1_hbm, gx2_hbm, gy2_hbm, labels_hbm,
                  rx1_hbm, ry1_hbm, rx2_hbm, ry2_hbm,
                  tx1_hbm, ty1_hbm, tx2_hbm, ty2_hbm,
                  lblout_hbm, isfg_hbm,
                  kf, ki, idx_a, idx_b, hist, offs, carry16,
                  idx_sel, am_v, gti_v, val_v, lbl_v, isfg_v, sem):
    c = lax.axis_index("c")
    s = lax.axis_index("s")
    wid = s * 2 + c
    active = wid < 16
    img = wid // 2
    cl = wid % 2

    @pl.when(active)
    def _():
        iota16 = lax.iota(jnp.int32, 16)
        lane15 = jnp.full((16,), 15, jnp.int32)

        pltpu.sync_copy(miou_hbm.at[pl.ds(img * NP, NP)], kf)

        def key_body(i, _):
            m = kf[pl.ds(i * 16, 16)]
            valid = (iota16 + i * 16) < NREAL
            fg = jnp.where(valid & (m >= FG_THRESH), m, -1.0)
            bg = jnp.where(valid & (m < FG_THRESH), 1.0 - m, -1.0)
            score = jnp.where(cl == 0, fg, bg)
            sb = lax.bitcast_convert_type(score, jnp.int32)
            inv = jnp.where(sb >= 0, _KEY_BIAS - sb, _INVALID24)
            kf[pl.ds(i * 16, 16)] = lax.bitcast_convert_type(inv, jnp.float32)
            return 0

        lax.fori_loop(0, NP // 16, key_body, 0)

        def radix_pass(shift, src_key, src_f32, dst_key, dst_f32,
                       src_idx, dst_idx, first):
            def load_key(i):
                k = src_key[pl.ds(i * 16, 16)]
                return lax.bitcast_convert_type(k, jnp.int32) if src_f32 else k

            zero16 = jnp.zeros((16,), jnp.int32)

            def z_body(j, _):
                hist[pl.ds(j * 16, 16)] = zero16
                return 0

            lax.fori_loop(0, NBKT // 16, z_body, 0)

            shift_v = jnp.full((16,), shift, jnp.int32)

            def h_body(i, _):
                kb = load_key(i)
                d = lax.shift_right_logical(kb, shift_v) & jnp.int32(NBKT - 1)
                cnt, last = plsc.scan_count(d)
                plsc.addupdate_scatter(hist, [d], cnt, mask=last)
                return 0

            lax.fori_loop(0, NP // 16, h_body, 0)

            def p_body(j, carry):
                v = hist[pl.ds(j * 16, 16)]
                csum = plsc.cumsum(v)
                offs[pl.ds(j * 16, 16)] = csum - v + carry
                carry16[...] = csum
                tot = plsc.load_gather(carry16, [lane15])
                return carry + tot

            lax.fori_loop(0, NBKT // 16, p_body, jnp.zeros((16,), jnp.int32))

            def s_body(i, _):
                kb = load_key(i)
                d = lax.shift_right_logical(kb, shift_v) & jnp.int32(NBKT - 1)
                if first:
                    iv = iota16 + i * 16
                else:
                    iv = src_idx[pl.ds(i * 16, 16)]
                base = plsc.load_gather(offs, [d])
                cnt, last = plsc.scan_count(d)
                pos = base + cnt - 1
                kout = lax.bitcast_convert_type(kb, jnp.float32) if dst_f32 else kb
                plsc.store_scatter(dst_key, [pos], kout)
                plsc.store_scatter(dst_idx, [pos], iv)
                plsc.addupdate_scatter(offs, [d], cnt, mask=last)
                return 0

            lax.fori_loop(0, NP // 16, s_body, 0)

        radix_pass(0, kf, True, ki, False, idx_a, idx_b, True)
        radix_pass(12, ki, False, kf, True, idx_b, idx_a, False)

        def emit(k_count, slot0, is_fg_unit):
            nv = k_count // 16
            nch = k_count // 128

            def sel_body(i, _):
                iv = idx_a[pl.ds(i * 16, 16)]
                row = i >> 3
                col = (i & 7) * 16
                idx_sel[row, pl.ds(col, 16)] = iv + img * NP
                if is_fg_unit:
                    inv = lax.bitcast_convert_type(kf[pl.ds(i * 16, 16)],
                                                   jnp.int32)
                    fgm = inv < _INVALID24
                    isfg_v[pl.ds(i * 16, 16)] = jnp.where(fgm, 1, 0)
                else:
                    isfg_v[pl.ds(i * 16, 16)] = jnp.zeros((16,), jnp.int32)
                return 0

            lax.fori_loop(0, nv, sel_body, 0)

            out0 = img * NSEL + slot0

            for j in range(nch):
                pltpu.async_copy(amax_hbm.at[idx_sel.at[j]],
                                 am_v.at[j], sem).wait()

            def gti_body(i, _):
                row = i >> 3
                col = (i & 7) * 16
                am = am_v[row, pl.ds(col, 16)]
                gti_v[row, pl.ds(col, 16)] = am + img * G
                return 0

            lax.fori_loop(0, nv, gti_body, 0)

            for src, dst in ((bx1_hbm, rx1_hbm), (by1_hbm, ry1_hbm),
                             (bx2_hbm, rx2_hbm), (by2_hbm, ry2_hbm)):
                for j in range(nch):
                    pltpu.async_copy(src.at[idx_sel.at[j]],
                                     val_v.at[pl.ds(j * 128, 128)], sem).wait()
                pltpu.sync_copy(val_v.at[pl.ds(0, k_count)],
                                dst.at[pl.ds(out0, k_count)])
            for src, dst in ((gx1_hbm, tx1_hbm), (gy1_hbm, ty1_hbm),
                             (gx2_hbm, tx2_hbm), (gy2_hbm, ty2_hbm)):
                for j in range(nch):
                    pltpu.async_copy(src.at[gti_v.at[j]],
                                     val_v.at[pl.ds(j * 128, 128)], sem).wait()
                pltpu.sync_copy(val_v.at[pl.ds(0, k_count)],
                                dst.at[pl.ds(out0, k_count)])
            for j in range(nch):
                pltpu.async_copy(labels_hbm.at[gti_v.at[j]],
                                 lbl_v.at[pl.ds(j * 128, 128)], sem).wait()

            pltpu.sync_copy(lbl_v.at[pl.ds(0, k_count)],
                            lblout_hbm.at[pl.ds(out0, k_count)])
            pltpu.sync_copy(isfg_v.at[pl.ds(0, k_count)],
                            isfg_hbm.at[pl.ds(out0, k_count)])

        @pl.when(cl == 0)
        def _():
            emit(NUM_FG, 0, True)

        @pl.when(cl == 1)
        def _():
            emit(NUM_BG, NUM_FG, False)


def _stage_b(miou_flat, amax_flat, box_planes, gt_planes, labels_flat):
    mesh = plsc.VectorSubcoreMesh(core_axis_name="c", subcore_axis_name="s")
    fvec = jax.ShapeDtypeStruct((B * NSEL,), jnp.float32)
    ivec = jax.ShapeDtypeStruct((B * NSEL,), jnp.int32)
    f = pl.kernel(
        _stage_b_body,
        out_type=[fvec, fvec, fvec, fvec,
                  fvec, fvec, fvec, fvec,
                  ivec, ivec],
        mesh=mesh,
        compiler_params=pltpu.CompilerParams(needs_layout_passes=False),
        scratch_types=[
            pltpu.VMEM((NP,), jnp.float32),
            pltpu.VMEM((NP,), jnp.int32),
            pltpu.VMEM((NP,), jnp.int32),
            pltpu.VMEM((NP,), jnp.int32),
            pltpu.VMEM((NBKT,), jnp.int32),
            pltpu.VMEM((NBKT,), jnp.int32),
            pltpu.VMEM((16,), jnp.int32),
            pltpu.VMEM((3, 128), jnp.int32),
            pltpu.VMEM((3, 128), jnp.int32),
            pltpu.VMEM((3, 128), jnp.int32),
            pltpu.VMEM((384,), jnp.float32),
            pltpu.VMEM((384,), jnp.int32),
            pltpu.VMEM((384,), jnp.int32),
            pltpu.SemaphoreType.DMA,
        ],
    )
    return f(miou_flat, amax_flat, *box_planes, *gt_planes, labels_flat)


def _stage_c_body(roist_ref, gtbt_ref, lbl_ref, isfg_ref, btt_ref, cls_ref):
    rx1 = roist_ref[0]
    ry1 = roist_ref[1]
    rx2 = roist_ref[2]
    ry2 = roist_ref[3]
    gx1 = gtbt_ref[0]
    gy1 = gtbt_ref[1]
    gx2 = gtbt_ref[2]
    gy2 = gtbt_ref[3]
    pw = jnp.maximum(rx2 - rx1, 1e-8)
    ph = jnp.maximum(ry2 - ry1, 1e-8)
    px = rx1 + 0.5 * pw
    py = ry1 + 0.5 * ph
    gw = jnp.maximum(gx2 - gx1, 1e-8)
    gh = jnp.maximum(gy2 - gy1, 1e-8)
    gx = gx1 + 0.5 * gw
    gy = gy1 + 0.5 * gh
    fg = isfg_ref[...] != 0
    zero = jnp.zeros_like(pw)
    btt_ref[0] = jnp.where(fg, (gx - px) / pw, zero)
    btt_ref[1] = jnp.where(fg, (gy - py) / ph, zero)
    btt_ref[2] = jnp.where(fg, jnp.log(gw / pw), zero)
    btt_ref[3] = jnp.where(fg, jnp.log(gh / ph), zero)
    cls_ref[...] = jnp.where(fg, lbl_ref[...], 0)


def _stage_c(rois_t, gtb_t, lbl, isfg):
    return pl.pallas_call(
        _stage_c_body,
        out_shape=[
            jax.ShapeDtypeStruct((4, 32, 128), jnp.float32),
            jax.ShapeDtypeStruct((32, 128), jnp.int32),
        ],
    )(rois_t, gtb_t, lbl, isfg)


@jax.jit
def kernel(boxes, gt_boxes, gt_labels):
    boxes = lax.stop_gradient(boxes)
    all_boxes = jnp.concatenate([boxes, gt_boxes], axis=1)
    ab_pad = jnp.pad(all_boxes, ((0, 0), (0, NP - NREAL), (0, 0)))
    planes = ab_pad.transpose(0, 2, 1).reshape(B, 4, NSUB, 128)
    x1 = planes[:, 0]
    y1 = planes[:, 1]
    x2 = planes[:, 2]
    y2 = planes[:, 3]

    miou, amax = _stage_a(gt_boxes, x1, y1, x2, y2)

    box_planes = [planes[:, c].reshape(B * NP) for c in range(4)]
    gt_planes = [gt_boxes[:, :, c].reshape(B * G) for c in range(4)]
    outs = _stage_b(
        miou.reshape(B * NP),
        amax.reshape(B * NP),
        box_planes,
        gt_planes,
        gt_labels.reshape(B * G),
    )
    rois_p = outs[0:4]
    gtb_p = outs[4:8]
    lbl_f, isfg_f = outs[8], outs[9]

    rois_t = jnp.stack(rois_p).reshape(4, 32, 128)
    gtb_t = jnp.stack(gtb_p).reshape(4, 32, 128)
    btt, cls = _stage_c(rois_t, gtb_t,
                        lbl_f.reshape(32, 128), isfg_f.reshape(32, 128))

    rois = jnp.stack(rois_p, axis=-1).reshape(B, NSEL, 4)
    box_targets = btt.reshape(4, B * NSEL).T.reshape(B, NSEL, 4)
    class_targets = cls.reshape(B, NSEL)
    return rois, box_targets, class_targets


# speedup vs baseline: 1.3256x; 1.0333x over previous
import functools

import numpy as np

import jax
import jax.numpy as jnp
from jax import lax
from jax.experimental import pallas as pl
from jax.experimental.pallas import tpu as pltpu
from jax.experimental.pallas import tpu_sc as plsc

B = 8
N = 20000
G = 100
NREAL = N + G
NP = 20480
NSUB = NP // 128
NUM_FG = 128
NUM_BG = 384
NSEL = NUM_FG + NUM_BG
FG_THRESH = 0.5
NBKT = 4096

_KEY_BIAS = np.int32(0x3F810000)
_INVALID24 = np.int32(0x00FFFFFF)


def _stage_a_body(gt_ref, x1_ref, y1_ref, x2_ref, y2_ref, miou_ref, amax_ref):
    x1 = x1_ref[0]
    y1 = y1_ref[0]
    x2 = x2_ref[0]
    y2 = y2_ref[0]
    area = (x2 - x1) * (y2 - y1)

    def body(g, carry):
        best, bidx = carry
        gx1 = gt_ref[0, g, 0]
        gy1 = gt_ref[0, g, 1]
        gx2 = gt_ref[0, g, 2]
        gy2 = gt_ref[0, g, 3]
        garea = (gx2 - gx1) * (gy2 - gy1)
        iw = jnp.maximum(jnp.minimum(x2, gx2) - jnp.maximum(x1, gx1), 0.0)
        ih = jnp.maximum(jnp.minimum(y2, gy2) - jnp.maximum(y1, gy1), 0.0)
        inter = iw * ih
        union = jnp.maximum(area + garea - inter, 1e-8)
        q = inter / union
        upd = q > best
        best = jnp.where(upd, q, best)
        bidx = jnp.where(upd, g, bidx)
        return best, bidx

    best0 = jnp.zeros_like(x1)
    bidx0 = jnp.zeros(x1.shape, jnp.int32)
    best, bidx = lax.fori_loop(0, G, body, (best0, bidx0), unroll=4)
    miou_ref[0] = best
    amax_ref[0] = bidx


def _stage_a(gt_boxes, x1, y1, x2, y2):
    chunk = 160
    grid = (B, NSUB // chunk)
    plane_spec = pl.BlockSpec((1, chunk, 128), lambda b, c: (b, c, 0))
    return pl.pallas_call(
        _stage_a_body,
        grid=grid,
        in_specs=[
            pl.BlockSpec((1, G, 4), lambda b, c: (b, 0, 0),
                         memory_space=pltpu.SMEM),
            plane_spec, plane_spec, plane_spec, plane_spec,
        ],
        out_specs=[plane_spec, plane_spec],
        out_shape=[
            jax.ShapeDtypeStruct((B, NSUB, 128), jnp.float32),
            jax.ShapeDtypeStruct((B, NSUB, 128), jnp.int32),
        ],
    )(gt_boxes, x1, y1, x2, y2)


def _stage_b_body(miou_hbm, amax_hbm,
                  bx1_hbm, by1_hbm, bx2_hbm, by2_hbm,
                  gx1_hbm, gy1_hbm, gx2_hbm, gy2_hbm, labels_hbm,
                  rx1_hbm, ry1_hbm, rx2_hbm, ry2_hbm,
                  tx1_hbm, ty1_hbm, tx2_hbm, ty2_hbm,
                  lblout_hbm, isfg_hbm,
                  kf, ki, idx_a, idx_b, hist, offs, carry16,
                  idx_sel, am_v, gti_v, val_v, lbl_v, isfg_v, sem):
    c = lax.axis_index("c")
    s = lax.axis_index("s")
    wid = s * 2 + c
    active = wid < 16
    img = wid // 2
    cl = wid % 2

    @pl.when(active)
    def _():
        iota16 = lax.iota(jnp.int32, 16)
        lane15 = jnp.full((16,), 15, jnp.int32)

        pltpu.sync_copy(miou_hbm.at[pl.ds(img * NP, NP)], kf)

        def key_body(i, _):
            m = kf[pl.ds(i * 16, 16)]
            valid = (iota16 + i * 16) < NREAL
            fg = jnp.where(valid & (m >= FG_THRESH), m, -1.0)
            bg = jnp.where(valid & (m < FG_THRESH), 1.0 - m, -1.0)
            score = jnp.where(cl == 0, fg, bg)
            sb = lax.bitcast_convert_type(score, jnp.int32)
            inv = jnp.where(sb >= 0, _KEY_BIAS - sb, _INVALID24)
            kf[pl.ds(i * 16, 16)] = lax.bitcast_convert_type(inv, jnp.float32)
            return 0

        lax.fori_loop(0, NP // 16, key_body, 0)

        def radix_pass(shift, src_key, src_f32, dst_key, dst_f32,
                       src_idx, dst_idx, first):
            def load_key(i):
                k = src_key[pl.ds(i * 16, 16)]
                return lax.bitcast_convert_type(k, jnp.int32) if src_f32 else k

            zero16 = jnp.zeros((16,), jnp.int32)

            def z_body(j, _):
                hist[pl.ds(j * 16, 16)] = zero16
                return 0

            lax.fori_loop(0, NBKT // 16, z_body, 0)

            shift_v = jnp.full((16,), shift, jnp.int32)

            def h_body(i, _):
                kb = load_key(i)
                d = lax.shift_right_logical(kb, shift_v) & jnp.int32(NBKT - 1)
                cnt, last = plsc.scan_count(d)
                plsc.addupdate_scatter(hist, [d], cnt, mask=last)
                return 0

            lax.fori_loop(0, NP // 16, h_body, 0)

            def p_body(j, carry):
                v = hist[pl.ds(j * 16, 16)]
                csum = plsc.cumsum(v)
                offs[pl.ds(j * 16, 16)] = csum - v + carry
                carry16[...] = csum
                tot = plsc.load_gather(carry16, [lane15])
                return carry + tot

            lax.fori_loop(0, NBKT // 16, p_body, jnp.zeros((16,), jnp.int32))

            def s_body(i, _):
                kb = load_key(i)
                d = lax.shift_right_logical(kb, shift_v) & jnp.int32(NBKT - 1)
                if first:
                    iv = iota16 + i * 16
                else:
                    iv = src_idx[pl.ds(i * 16, 16)]
                base = plsc.load_gather(offs, [d])
                cnt, last = plsc.scan_count(d)
                pos = base + cnt - 1
                kout = lax.bitcast_convert_type(kb, jnp.float32) if dst_f32 else kb
                plsc.store_scatter(dst_key, [pos], kout)
                plsc.store_scatter(dst_idx, [pos], iv)
                plsc.addupdate_scatter(offs, [d], cnt, mask=last)
                return 0

            lax.fori_loop(0, NP // 16, s_body, 0)

        radix_pass(0, kf, True, ki, False, idx_a, idx_b, True)
        radix_pass(12, ki, False, kf, True, idx_b, idx_a, False)

        def emit(k_count, slot0, is_fg_unit):
            nv = k_count // 16
            nch = k_count // 128

            def sel_body(i, _):
                iv = idx_a[pl.ds(i * 16, 16)]
                row = i >> 3
                col = (i & 7) * 16
                idx_sel[row, pl.ds(col, 16)] = iv + img * NP
                if is_fg_unit:
                    inv = lax.bitcast_convert_type(kf[pl.ds(i * 16, 16)],
                                                   jnp.int32)
                    fgm = inv < _INVALID24
                    isfg_v[pl.ds(i * 16, 16)] = jnp.where(fgm, 1, 0)
                else:
                    isfg_v[pl.ds(i * 16, 16)] = jnp.zeros((16,), jnp.int32)
                return 0

            lax.fori_loop(0, nv, sel_body, 0)

            out0 = img * NSEL + slot0

            for j in range(nch):
                pltpu.async_copy(amax_hbm.at[idx_sel.at[j]],
                                 am_v.at[j], sem).wait()

            def gti_body(i, _):
                row = i >> 3
                col = (i & 7) * 16
                am = am_v[row, pl.ds(col, 16)]
                gti_v[row, pl.ds(col, 16)] = am + img * G
                return 0

            lax.fori_loop(0, nv, gti_body, 0)

            for src, dst in ((bx1_hbm, rx1_hbm), (by1_hbm, ry1_hbm),
                             (bx2_hbm, rx2_hbm), (by2_hbm, ry2_hbm)):
                for j in range(nch):
                    pltpu.async_copy(src.at[idx_sel.at[j]],
                                     val_v.at[pl.ds(j * 128, 128)], sem).wait()
                pltpu.sync_copy(val_v.at[pl.ds(0, k_count)],
                                dst.at[pl.ds(out0, k_count)])
            for src, dst in ((gx1_hbm, tx1_hbm), (gy1_hbm, ty1_hbm),
                             (gx2_hbm, tx2_hbm), (gy2_hbm, ty2_hbm)):
                for j in range(nch):
                    pltpu.async_copy(src.at[gti_v.at[j]],
                                     val_v.at[pl.ds(j * 128, 128)], sem).wait()
                pltpu.sync_copy(val_v.at[pl.ds(0, k_count)],
                                dst.at[pl.ds(out0, k_count)])
            for j in range(nch):
                pltpu.async_copy(labels_hbm.at[gti_v.at[j]],
                                 lbl_v.at[pl.ds(j * 128, 128)], sem).wait()

            pltpu.sync_copy(lbl_v.at[pl.ds(0, k_count)],
                            lblout_hbm.at[pl.ds(out0, k_count)])
            pltpu.sync_copy(isfg_v.at[pl.ds(0, k_count)],
                            isfg_hbm.at[pl.ds(out0, k_count)])

        @pl.when(cl == 0)
        def _():
            emit(NUM_FG, 0, True)

        @pl.when(cl == 1)
        def _():
            emit(NUM_BG, NUM_FG, False)


def _stage_b(miou_flat, amax_flat, box_planes, gt_planes, labels_flat):
    mesh = plsc.VectorSubcoreMesh(core_axis_name="c", subcore_axis_name="s")
    fvec = jax.ShapeDtypeStruct((B * NSEL,), jnp.float32)
    ivec = jax.ShapeDtypeStruct((B * NSEL,), jnp.int32)
    f = pl.kernel(
        _stage_b_body,
        out_type=[fvec, fvec, fvec, fvec,
                  fvec, fvec, fvec, fvec,
                  ivec, ivec],
        mesh=mesh,
        compiler_params=pltpu.CompilerParams(needs_layout_passes=False),
        scratch_types=[
            pltpu.VMEM((NP,), jnp.float32),
            pltpu.VMEM((NP,), jnp.int32),
            pltpu.VMEM((NP,), jnp.int32),
            pltpu.VMEM((NP,), jnp.int32),
            pltpu.VMEM((NBKT,), jnp.int32),
            pltpu.VMEM((NBKT,), jnp.int32),
            pltpu.VMEM((16,), jnp.int32),
            pltpu.VMEM((3, 128), jnp.int32),
            pltpu.VMEM((3, 128), jnp.int32),
            pltpu.VMEM((3, 128), jnp.int32),
            pltpu.VMEM((384,), jnp.float32),
            pltpu.VMEM((384,), jnp.int32),
            pltpu.VMEM((384,), jnp.int32),
            pltpu.SemaphoreType.DMA,
        ],
    )
    return f(miou_flat, amax_flat, *box_planes, *gt_planes, labels_flat)


def _stage_c_body(roist_ref, gtbt_ref, lbl_ref, isfg_ref, btt_ref, cls_ref):
    rx1 = roist_ref[0]
    ry1 = roist_ref[1]
    rx2 = roist_ref[2]
    ry2 = roist_ref[3]
    gx1 = gtbt_ref[0]
    gy1 = gtbt_ref[1]
    gx2 = gtbt_ref[2]
    gy2 = gtbt_ref[3]
    pw = jnp.maximum(rx2 - rx1, 1e-8)
    ph = jnp.maximum(ry2 - ry1, 1e-8)
    px = rx1 + 0.5 * pw
    py = ry1 + 0.5 * ph
    gw = jnp.maximum(gx2 - gx1, 1e-8)
    gh = jnp.maximum(gy2 - gy1, 1e-8)
    gx = gx1 + 0.5 * gw
    gy = gy1 + 0.5 * gh
    fg = isfg_ref[...] != 0
    zero = jnp.zeros_like(pw)
    btt_ref[0] = jnp.where(fg, (gx - px) / pw, zero)
    btt_ref[1] = jnp.where(fg, (gy - py) / ph, zero)
    btt_ref[2] = jnp.where(fg, jnp.log(gw / pw), zero)
    btt_ref[3] = jnp.where(fg, jnp.log(gh / ph), zero)
    cls_ref[...] = jnp.where(fg, lbl_ref[...], 0)


def _stage_c(rois_t, gtb_t, lbl, isfg):
    return pl.pallas_call(
        _stage_c_body,
        out_shape=[
            jax.ShapeDtypeStruct((4, 32, 128), jnp.float32),
            jax.ShapeDtypeStruct((32, 128), jnp.int32),
        ],
    )(rois_t, gtb_t, lbl, isfg)


@jax.jit
def kernel(boxes, gt_boxes, gt_labels):
    boxes = lax.stop_gradient(boxes)
    all_boxes = jnp.concatenate([boxes, gt_boxes], axis=1)
    ab_pad = jnp.pad(all_boxes, ((0, 0), (0, NP - NREAL), (0, 0)))
    planes = ab_pad.transpose(0, 2, 1).reshape(B, 4, NSUB, 128)
    x1 = planes[:, 0]
    y1 = planes[:, 1]
    x2 = planes[:, 2]
    y2 = planes[:, 3]

    miou, amax = _stage_a(gt_boxes, x1, y1, x2, y2)

    box_planes = [planes[:, c].reshape(B * NP) for c in range(4)]
    gt_planes = [gt_boxes[:, :, c].reshape(B * G) for c in range(4)]
    outs = _stage_b(
        miou.reshape(B * NP),
        amax.reshape(B * NP),
        box_planes,
        gt_planes,
        gt_labels.reshape(B * G),
    )
    rois_p = outs[0:4]
    gtb_p = outs[4:8]
    lbl_f, isfg_f = outs[8], outs[9]

    rois_t = jnp.stack(rois_p).reshape(4, 32, 128)
    gtb_t = jnp.stack(gtb_p).reshape(4, 32, 128)
    btt, cls = _stage_c(rois_t, gtb_t,
                        lbl_f.reshape(32, 128), isfg_f.reshape(32, 128))

    rois = jnp.stack(rois_p, axis=-1).reshape(B, NSEL, 4)
    box_targets = btt.reshape(4, B * NSEL).T.reshape(B, NSEL, 4)
    class_targets = cls.reshape(B, NSEL)
    return rois, box_targets, class_targets


# speedup vs baseline: 1.3456x; 1.0150x over previous
import functools

import numpy as np

import jax
import jax.numpy as jnp
from jax import lax
from jax.experimental import pallas as pl
from jax.experimental.pallas import tpu as pltpu
from jax.experimental.pallas import tpu_sc as plsc

B = 8
N = 20000
G = 100
NREAL = N + G
NP = 20480
NSUB = NP // 128
NUM_FG = 128
NUM_BG = 384
NSEL = NUM_FG + NUM_BG
FG_THRESH = 0.5
NBKT = 4096
NVEC = (NREAL + 15) // 16

_KEY_BIAS = np.int32(0x3F810000)
_INVALID24 = np.int32(0x00FFFFFF)


def _stage_a_body(gt_ref, x1_ref, y1_ref, x2_ref, y2_ref, miou_ref, amax_ref):
    x1 = x1_ref[0]
    y1 = y1_ref[0]
    x2 = x2_ref[0]
    y2 = y2_ref[0]
    area = (x2 - x1) * (y2 - y1)

    def body(g, carry):
        best, bidx = carry
        gx1 = gt_ref[0, g, 0]
        gy1 = gt_ref[0, g, 1]
        gx2 = gt_ref[0, g, 2]
        gy2 = gt_ref[0, g, 3]
        garea = (gx2 - gx1) * (gy2 - gy1)
        iw = jnp.maximum(jnp.minimum(x2, gx2) - jnp.maximum(x1, gx1), 0.0)
        ih = jnp.maximum(jnp.minimum(y2, gy2) - jnp.maximum(y1, gy1), 0.0)
        inter = iw * ih
        union = jnp.maximum(area + garea - inter, 1e-8)
        q = inter / union
        upd = q > best
        best = jnp.where(upd, q, best)
        bidx = jnp.where(upd, g, bidx)
        return best, bidx

    best0 = jnp.zeros_like(x1)
    bidx0 = jnp.zeros(x1.shape, jnp.int32)
    best, bidx = lax.fori_loop(0, G, body, (best0, bidx0), unroll=10)
    miou_ref[0] = best
    amax_ref[0] = bidx


def _stage_a(gt_boxes, x1, y1, x2, y2):
    chunk = 160
    grid = (B, NSUB // chunk)
    plane_spec = pl.BlockSpec((1, chunk, 128), lambda b, c: (b, c, 0))
    return pl.pallas_call(
        _stage_a_body,
        grid=grid,
        in_specs=[
            pl.BlockSpec((1, G, 4), lambda b, c: (b, 0, 0),
                         memory_space=pltpu.SMEM),
            plane_spec, plane_spec, plane_spec, plane_spec,
        ],
        out_specs=[plane_spec, plane_spec],
        out_shape=[
            jax.ShapeDtypeStruct((B, NSUB, 128), jnp.float32),
            jax.ShapeDtypeStruct((B, NSUB, 128), jnp.int32),
        ],
    )(gt_boxes, x1, y1, x2, y2)


def _stage_b_body(miou_hbm, amax_hbm,
                  bx1_hbm, by1_hbm, bx2_hbm, by2_hbm,
                  gx1_hbm, gy1_hbm, gx2_hbm, gy2_hbm, labels_hbm,
                  rx1_hbm, ry1_hbm, rx2_hbm, ry2_hbm,
                  tx1_hbm, ty1_hbm, tx2_hbm, ty2_hbm,
                  lblout_hbm, isfg_hbm,
                  kf, ki, idx_a, idx_b, hist, offs, carry16,
                  idx_sel, am_v, gti_v, val_v, lbl_v, isfg_v, sem):
    c = lax.axis_index("c")
    s = lax.axis_index("s")
    wid = s * 2 + c
    active = wid < 16
    img = wid // 2
    cl = wid % 2

    @pl.when(active)
    def _():
        iota16 = lax.iota(jnp.int32, 16)
        lane15 = jnp.full((16,), 15, jnp.int32)

        pltpu.sync_copy(miou_hbm.at[pl.ds(img * NP, NP)], kf)

        def key_body(i, _):
            m = kf[pl.ds(i * 16, 16)]
            valid = (iota16 + i * 16) < NREAL
            fg = jnp.where(valid & (m >= FG_THRESH), m, -1.0)
            bg = jnp.where(valid & (m < FG_THRESH), 1.0 - m, -1.0)
            score = jnp.where(cl == 0, fg, bg)
            sb = lax.bitcast_convert_type(score, jnp.int32)
            inv = jnp.where(sb >= 0, _KEY_BIAS - sb, _INVALID24)
            kf[pl.ds(i * 16, 16)] = lax.bitcast_convert_type(inv, jnp.float32)
            return 0

        lax.fori_loop(0, NVEC, key_body, 0)

        def radix_pass(shift, src_key, src_f32, dst_key, dst_f32,
                       src_idx, dst_idx, first):
            def load_key(i):
                k = src_key[pl.ds(i * 16, 16)]
                return lax.bitcast_convert_type(k, jnp.int32) if src_f32 else k

            zero16 = jnp.zeros((16,), jnp.int32)

            def z_body(j, _):
                hist[pl.ds(j * 16, 16)] = zero16
                return 0

            lax.fori_loop(0, NBKT // 16, z_body, 0)

            shift_v = jnp.full((16,), shift, jnp.int32)

            def h_body(i, _):
                kb = load_key(i)
                d = lax.shift_right_logical(kb, shift_v) & jnp.int32(NBKT - 1)
                cnt, last = plsc.scan_count(d)
                plsc.addupdate_scatter(hist, [d], cnt, mask=last)
                return 0

            lax.fori_loop(0, NVEC, h_body, 0)

            def p_body(j, carry):
                v = hist[pl.ds(j * 16, 16)]
                csum = plsc.cumsum(v)
                offs[pl.ds(j * 16, 16)] = csum - v + carry
                carry16[...] = csum
                tot = plsc.load_gather(carry16, [lane15])
                return carry + tot

            lax.fori_loop(0, NBKT // 16, p_body, jnp.zeros((16,), jnp.int32))

            def s_body(i, _):
                kb = load_key(i)
                d = lax.shift_right_logical(kb, shift_v) & jnp.int32(NBKT - 1)
                if first:
                    iv = iota16 + i * 16
                else:
                    iv = src_idx[pl.ds(i * 16, 16)]
                base = plsc.load_gather(offs, [d])
                cnt, last = plsc.scan_count(d)
                pos = base + cnt - 1
                kout = lax.bitcast_convert_type(kb, jnp.float32) if dst_f32 else kb
                plsc.store_scatter(dst_key, [pos], kout)
                plsc.store_scatter(dst_idx, [pos], iv)
                plsc.addupdate_scatter(offs, [d], cnt, mask=last)
                return 0

            lax.fori_loop(0, NVEC, s_body, 0)

        radix_pass(0, kf, True, ki, False, idx_a, idx_b, True)
        radix_pass(12, ki, False, kf, True, idx_b, idx_a, False)

        def emit(k_count, slot0, is_fg_unit):
            nv = k_count // 16
            nch = k_count // 128

            def sel_body(i, _):
                iv = idx_a[pl.ds(i * 16, 16)]
                row = i >> 3
                col = (i & 7) * 16
                idx_sel[row, pl.ds(col, 16)] = iv + img * NP
                if is_fg_unit:
                    inv = lax.bitcast_convert_type(kf[pl.ds(i * 16, 16)],
                                                   jnp.int32)
                    fgm = inv < _INVALID24
                    isfg_v[pl.ds(i * 16, 16)] = jnp.where(fgm, 1, 0)
                else:
                    isfg_v[pl.ds(i * 16, 16)] = jnp.zeros((16,), jnp.int32)
                return 0

            lax.fori_loop(0, nv, sel_body, 0)

            out0 = img * NSEL + slot0

            for j in range(nch):
                pltpu.async_copy(amax_hbm.at[idx_sel.at[j]],
                                 am_v.at[j], sem).wait()

            def gti_body(i, _):
                row = i >> 3
                col = (i & 7) * 16
                am = am_v[row, pl.ds(col, 16)]
                gti_v[row, pl.ds(col, 16)] = am + img * G
                return 0

            lax.fori_loop(0, nv, gti_body, 0)

            for src, dst in ((bx1_hbm, rx1_hbm), (by1_hbm, ry1_hbm),
                             (bx2_hbm, rx2_hbm), (by2_hbm, ry2_hbm)):
                for j in range(nch):
                    pltpu.async_copy(src.at[idx_sel.at[j]],
                                     val_v.at[pl.ds(j * 128, 128)], sem).wait()
                pltpu.sync_copy(val_v.at[pl.ds(0, k_count)],
                                dst.at[pl.ds(out0, k_count)])
            for src, dst in ((gx1_hbm, tx1_hbm), (gy1_hbm, ty1_hbm),
                             (gx2_hbm, tx2_hbm), (gy2_hbm, ty2_hbm)):
                for j in range(nch):
                    pltpu.async_copy(src.at[gti_v.at[j]],
                                     val_v.at[pl.ds(j * 128, 128)], sem).wait()
                pltpu.sync_copy(val_v.at[pl.ds(0, k_count)],
                                dst.at[pl.ds(out0, k_count)])
            for j in range(nch):
                pltpu.async_copy(labels_hbm.at[gti_v.at[j]],
                                 lbl_v.at[pl.ds(j * 128, 128)], sem).wait()

            pltpu.sync_copy(lbl_v.at[pl.ds(0, k_count)],
                            lblout_hbm.at[pl.ds(out0, k_count)])
            pltpu.sync_copy(isfg_v.at[pl.ds(0, k_count)],
                            isfg_hbm.at[pl.ds(out0, k_count)])

        @pl.when(cl == 0)
        def _():
            emit(NUM_FG, 0, True)

        @pl.when(cl == 1)
        def _():
            emit(NUM_BG, NUM_FG, False)


def _stage_b(miou_flat, amax_flat, box_planes, gt_planes, labels_flat):
    mesh = plsc.VectorSubcoreMesh(core_axis_name="c", subcore_axis_name="s")
    fvec = jax.ShapeDtypeStruct((B * NSEL,), jnp.float32)
    ivec = jax.ShapeDtypeStruct((B * NSEL,), jnp.int32)
    f = pl.kernel(
        _stage_b_body,
        out_type=[fvec, fvec, fvec, fvec,
                  fvec, fvec, fvec, fvec,
                  ivec, ivec],
        mesh=mesh,
        compiler_params=pltpu.CompilerParams(needs_layout_passes=False),
        scratch_types=[
            pltpu.VMEM((NP,), jnp.float32),
            pltpu.VMEM((NP,), jnp.int32),
            pltpu.VMEM((NP,), jnp.int32),
            pltpu.VMEM((NP,), jnp.int32),
            pltpu.VMEM((NBKT,), jnp.int32),
            pltpu.VMEM((NBKT,), jnp.int32),
            pltpu.VMEM((16,), jnp.int32),
            pltpu.VMEM((3, 128), jnp.int32),
            pltpu.VMEM((3, 128), jnp.int32),
            pltpu.VMEM((3, 128), jnp.int32),
            pltpu.VMEM((384,), jnp.float32),
            pltpu.VMEM((384,), jnp.int32),
            pltpu.VMEM((384,), jnp.int32),
            pltpu.SemaphoreType.DMA,
        ],
    )
    return f(miou_flat, amax_flat, *box_planes, *gt_planes, labels_flat)


def _stage_c_body(roist_ref, gtbt_ref, lbl_ref, isfg_ref, btt_ref, cls_ref):
    rx1 = roist_ref[0]
    ry1 = roist_ref[1]
    rx2 = roist_ref[2]
    ry2 = roist_ref[3]
    gx1 = gtbt_ref[0]
    gy1 = gtbt_ref[1]
    gx2 = gtbt_ref[2]
    gy2 = gtbt_ref[3]
    pw = jnp.maximum(rx2 - rx1, 1e-8)
    ph = jnp.maximum(ry2 - ry1, 1e-8)
    px = rx1 + 0.5 * pw
    py = ry1 + 0.5 * ph
    gw = jnp.maximum(gx2 - gx1, 1e-8)
    gh = jnp.maximum(gy2 - gy1, 1e-8)
    gx = gx1 + 0.5 * gw
    gy = gy1 + 0.5 * gh
    fg = isfg_ref[...] != 0
    zero = jnp.zeros_like(pw)
    btt_ref[0] = jnp.where(fg, (gx - px) / pw, zero)
    btt_ref[1] = jnp.where(fg, (gy - py) / ph, zero)
    btt_ref[2] = jnp.where(fg, jnp.log(gw / pw), zero)
    btt_ref[3] = jnp.where(fg, jnp.log(gh / ph), zero)
    cls_ref[...] = jnp.where(fg, lbl_ref[...], 0)


def _stage_c(rois_t, gtb_t, lbl, isfg):
    return pl.pallas_call(
        _stage_c_body,
        out_shape=[
            jax.ShapeDtypeStruct((4, 32, 128), jnp.float32),
            jax.ShapeDtypeStruct((32, 128), jnp.int32),
        ],
    )(rois_t, gtb_t, lbl, isfg)


@jax.jit
def kernel(boxes, gt_boxes, gt_labels):
    boxes = lax.stop_gradient(boxes)
    all_boxes = jnp.concatenate([boxes, gt_boxes], axis=1)
    ab_pad = jnp.pad(all_boxes, ((0, 0), (0, NP - NREAL), (0, 0)))
    planes = ab_pad.transpose(0, 2, 1).reshape(B, 4, NSUB, 128)
    x1 = planes[:, 0]
    y1 = planes[:, 1]
    x2 = planes[:, 2]
    y2 = planes[:, 3]

    miou, amax = _stage_a(gt_boxes, x1, y1, x2, y2)

    box_planes = [planes[:, c].reshape(B * NP) for c in range(4)]
    gt_planes = [gt_boxes[:, :, c].reshape(B * G) for c in range(4)]
    outs = _stage_b(
        miou.reshape(B * NP),
        amax.reshape(B * NP),
        box_planes,
        gt_planes,
        gt_labels.reshape(B * G),
    )
    rois_p = outs[0:4]
    gtb_p = outs[4:8]
    lbl_f, isfg_f = outs[8], outs[9]

    rois_t = jnp.stack(rois_p).reshape(4, 32, 128)
    gtb_t = jnp.stack(gtb_p).reshape(4, 32, 128)
    btt, cls = _stage_c(rois_t, gtb_t,
                        lbl_f.reshape(32, 128), isfg_f.reshape(32, 128))

    rois = jnp.stack(rois_p, axis=-1).reshape(B, NSEL, 4)
    box_targets = btt.reshape(4, B * NSEL).T.reshape(B, NSEL, 4)
    class_targets = cls.reshape(B, NSEL)
    return rois, box_targets, class_targets
